# Initial kernel scaffold; baseline (speedup 1.0000x reference)
import jax
import jax.numpy as jnp
from jax.experimental import pallas as pl


def kernel(student_logits, teacher_logits, label_mask):
    raise NotImplementedError("write your pallas kernel here")


# speedup vs baseline: 16.4500x; 16.4500x over previous
import functools

import jax
import jax.numpy as jnp
from jax.experimental import pallas as pl

_K = 256
_EPS = 1e-8
_BISECT_ITERS = 22


def _sdsp_body(s_ref, t_ref, o_ref):
    S = s_ref[...]
    T = t_ref[...]

    m_s = jnp.max(S, axis=1, keepdims=True)
    mn_s = jnp.min(S, axis=1, keepdims=True)
    m_t = jnp.max(T, axis=1, keepdims=True)

    kf = jnp.float32(_K)

    def bisect_step(_, carry):
        lo, hi = carry
        mid = 0.5 * (lo + hi)
        cnt = jnp.sum((S >= mid).astype(jnp.float32), axis=1, keepdims=True)
        ge = cnt >= kf
        return jnp.where(ge, mid, lo), jnp.where(ge, hi, mid)

    lo, hi = jax.lax.fori_loop(
        0, _BISECT_ITERS, bisect_step, (mn_s, m_s + 1.0)
    )

    full = (S >= hi).astype(jnp.float32)
    band = ((S >= lo) & (S < hi)).astype(jnp.float32)
    cnt_hi = jnp.sum(full, axis=1, keepdims=True)
    nband = jnp.sum(band, axis=1, keepdims=True)
    need = kf - cnt_hi
    w = full + (need / jnp.maximum(nband, 1.0)) * band

    es = jnp.exp(S - m_s)
    et = jnp.exp(T - m_t)
    sumexp_s = jnp.sum(es, axis=1, keepdims=True)
    sumexp_t = jnp.sum(et, axis=1, keepdims=True)
    U = jnp.sum(w * es, axis=1, keepdims=True)
    V = jnp.sum(w * es * (S - T), axis=1, keepdims=True)
    W = jnp.sum(w * et, axis=1, keepdims=True)

    lse_s = m_s + jnp.log(jnp.maximum(sumexp_s, 1e-20))
    lse_t = m_t + jnp.log(jnp.maximum(sumexp_t, 1e-20))

    base_mass = U / sumexp_s
    cond_mass = W / sumexp_t
    topk_kl = V / sumexp_s + (lse_t - lse_s) * base_mass

    base_tail = jnp.maximum(1.0 - jnp.clip(base_mass, 0.0, 1.0 - _EPS), _EPS)
    cond_tail = jnp.maximum(1.0 - jnp.clip(cond_mass, 0.0, 1.0 - _EPS), _EPS)
    tail_kl = base_tail * (jnp.log(base_tail) - jnp.log(cond_tail))

    token_kl = topk_kl + tail_kl
    o_ref[...] = token_kl.reshape(1, 1, -1)


@functools.partial(jax.jit, static_argnames=())
def kernel(student_logits, teacher_logits, label_mask):
    n, v = student_logits.shape
    rows = 8
    grid = n // rows
    token_kl = pl.pallas_call(
        _sdsp_body,
        grid=(grid,),
        in_specs=[
            pl.BlockSpec((rows, v), lambda i: (i, 0)),
            pl.BlockSpec((rows, v), lambda i: (i, 0)),
        ],
        out_specs=pl.BlockSpec((1, 1, rows), lambda i: (i, 0, 0)),
        out_shape=jax.ShapeDtypeStruct((grid, 1, rows), jnp.float32),
    )(student_logits, teacher_logits)
    token_kl = token_kl.reshape(n)
    mask_f = label_mask.astype(jnp.float32)
    denom = jnp.maximum(jnp.sum(mask_f), 1.0)
    return jnp.sum(token_kl * mask_f) / denom


# speedup vs baseline: 23.5244x; 1.4301x over previous
import functools

import jax
import jax.numpy as jnp
from jax.experimental import pallas as pl

_K = 256
_EPS = 1e-8
_BISECT_ITERS = 14


def _sdsp_body(s_ref, t_ref, o_ref):
    S = s_ref[...]
    T = t_ref[...]

    m_s = jnp.max(S, axis=1, keepdims=True)
    mn_s = jnp.min(S, axis=1, keepdims=True)
    m_t = jnp.max(T, axis=1, keepdims=True)

    kf = jnp.float32(_K)

    def bisect_step(_, carry):
        lo, hi = carry
        mid = 0.5 * (lo + hi)
        cnt = jnp.sum((S >= mid).astype(jnp.float32), axis=1, keepdims=True)
        ge = cnt >= kf
        return jnp.where(ge, mid, lo), jnp.where(ge, hi, mid)

    lo, hi = jax.lax.fori_loop(
        0, _BISECT_ITERS, bisect_step, (mn_s, m_s + 1.0)
    )

    full = (S >= hi).astype(jnp.float32)
    band = ((S >= lo) & (S < hi)).astype(jnp.float32)
    cnt_hi = jnp.sum(full, axis=1, keepdims=True)
    nband = jnp.sum(band, axis=1, keepdims=True)
    need = kf - cnt_hi
    w = full + (need / jnp.maximum(nband, 1.0)) * band

    es = jnp.exp(S - m_s)
    et = jnp.exp(T - m_t)
    sumexp_s = jnp.sum(es, axis=1, keepdims=True)
    sumexp_t = jnp.sum(et, axis=1, keepdims=True)
    U = jnp.sum(w * es, axis=1, keepdims=True)
    V = jnp.sum(w * es * (S - T), axis=1, keepdims=True)
    W = jnp.sum(w * et, axis=1, keepdims=True)

    lse_s = m_s + jnp.log(jnp.maximum(sumexp_s, 1e-20))
    lse_t = m_t + jnp.log(jnp.maximum(sumexp_t, 1e-20))

    base_mass = U / sumexp_s
    cond_mass = W / sumexp_t
    topk_kl = V / sumexp_s + (lse_t - lse_s) * base_mass

    base_tail = jnp.maximum(1.0 - jnp.clip(base_mass, 0.0, 1.0 - _EPS), _EPS)
    cond_tail = jnp.maximum(1.0 - jnp.clip(cond_mass, 0.0, 1.0 - _EPS), _EPS)
    tail_kl = base_tail * (jnp.log(base_tail) - jnp.log(cond_tail))

    token_kl = topk_kl + tail_kl
    o_ref[...] = token_kl.reshape(1, 1, -1)


@functools.partial(jax.jit, static_argnames=())
def kernel(student_logits, teacher_logits, label_mask):
    n, v = student_logits.shape
    rows = 8
    grid = n // rows
    token_kl = pl.pallas_call(
        _sdsp_body,
        grid=(grid,),
        in_specs=[
            pl.BlockSpec((rows, v), lambda i: (i, 0)),
            pl.BlockSpec((rows, v), lambda i: (i, 0)),
        ],
        out_specs=pl.BlockSpec((1, 1, rows), lambda i: (i, 0, 0)),
        out_shape=jax.ShapeDtypeStruct((grid, 1, rows), jnp.float32),
    )(student_logits, teacher_logits)
    token_kl = token_kl.reshape(n)
    mask_f = label_mask.astype(jnp.float32)
    denom = jnp.maximum(jnp.sum(mask_f), 1.0)
    return jnp.sum(token_kl * mask_f) / denom
